# baseline (speedup 1.0000x reference)
import functools

import jax
import jax.numpy as jnp
from jax import lax
from jax.experimental import pallas as pl
from jax.experimental.pallas import tpu as pltpu
from jax.experimental.pallas import tpu_sc as plsc

_HID = 64
_P_BLK = 4000
_N_BLK = 256

_NC = 2
_NS = 16
_NW = _NC * _NS
_CHUNK = 112


def _point_scan_body(x_ref, idx_ref, prm_ref, w2bd_ref, out_ref, cval_ref, cidx_ref):
    B = x_ref.shape[0]

    @pl.when(pl.program_id(0) == 0)
    def _():
        cval_ref[...] = jnp.full((1, 2 * _HID), -jnp.inf, jnp.float32)
        cidx_ref[...] = jnp.full((1, 2), -1, jnp.int32)

    def mlp1(c0, p):
        h = (x_ref[:, c0:c0 + 1] * p[0:1, :]
             + x_ref[:, c0 + 1:c0 + 2] * p[1:2, :]
             + x_ref[:, c0 + 2:c0 + 3] * p[2:3, :]
             + p[3:4, :])
        m = jnp.mean(h, axis=-1, keepdims=True)
        v = jnp.mean((h - m) ** 2, axis=-1, keepdims=True)
        h = (h - m) * lax.rsqrt(v + 1e-5) * p[4:5, :] + p[5:6, :]
        return jnp.maximum(h, 0.0)

    hcat = jnp.concatenate([mlp1(0, prm_ref[0]), mlp1(3, prm_ref[1])], axis=1)
    bcat = jnp.concatenate([prm_ref[0, 6:7, :], prm_ref[1, 6:7, :]], axis=1)
    h2 = jnp.dot(hcat, w2bd_ref[...], preferred_element_type=jnp.float32) + bcat
    h0, h1 = h2[:, :_HID], h2[:, _HID:]

    idx = idx_ref[...]
    i0, i1 = idx[:, 0:1], idx[:, 1:2]
    row0 = lax.broadcasted_iota(jnp.int32, (B, 1), 0) == 0
    h0 = jnp.where(row0 & (i0 == cidx_ref[:, 0:1]),
                   jnp.maximum(h0, cval_ref[:, :_HID]), h0)
    h1 = jnp.where(row0 & (i1 == cidx_ref[:, 1:2]),
                   jnp.maximum(h1, cval_ref[:, _HID:]), h1)

    G = 8
    NG = B // G
    h0r = h0.reshape(NG, G, _HID)
    h1r = h1.reshape(NG, G, _HID)
    i0r = i0.reshape(NG, G, 1)
    i1r = i1.reshape(NG, G, 1)

    def gshift(a, s, fill):
        return jnp.concatenate(
            [jnp.full((NG, s, a.shape[2]), fill, a.dtype), a[:, :G - s, :]], axis=1)

    for s in (1, 2, 4):
        h0r = jnp.where(i0r == gshift(i0r, s, -1),
                        jnp.maximum(h0r, gshift(h0r, s, -jnp.inf)), h0r)
        h1r = jnp.where(i1r == gshift(i1r, s, -1),
                        jnp.maximum(h1r, gshift(h1r, s, -jnp.inf)), h1r)

    S0 = h0r[:, G - 1, :]
    S1 = h1r[:, G - 1, :]
    L0 = i0r[:, G - 1, :]
    L1 = i1r[:, G - 1, :]

    def sshift(a, s, fill):
        return jnp.concatenate(
            [jnp.full((s, a.shape[1]), fill, a.dtype), a[:NG - s, :]], axis=0)

    s = 1
    while s < NG:
        S0 = jnp.where(L0 == sshift(L0, s, -1),
                       jnp.maximum(S0, sshift(S0, s, -jnp.inf)), S0)
        S1 = jnp.where(L1 == sshift(L1, s, -1),
                       jnp.maximum(S1, sshift(S1, s, -jnp.inf)), S1)
        s *= 2

    P0 = sshift(S0, 1, -jnp.inf)
    P1 = sshift(S1, 1, -jnp.inf)
    PI0 = sshift(L0, 1, -1)
    PI1 = sshift(L1, 1, -1)
    h0r = jnp.where(i0r == PI0[:, :, None],
                    jnp.maximum(h0r, P0[:, None, :]), h0r)
    h1r = jnp.where(i1r == PI1[:, :, None],
                    jnp.maximum(h1r, P1[:, None, :]), h1r)

    res = jnp.concatenate(
        [h0r.reshape(B, _HID), h1r.reshape(B, _HID)], axis=1)
    out_ref[...] = res
    cval_ref[...] = res[B - 1:B, :]
    cidx_ref[...] = idx[B - 1:B, :]


def _point_stage(x_cat, idx_cat, prm2, w2bd, P):
    return pl.pallas_call(
        _point_scan_body,
        grid=(P // _P_BLK,),
        in_specs=[
            pl.BlockSpec((_P_BLK, 6), lambda j: (j, 0)),
            pl.BlockSpec((_P_BLK, 2), lambda j: (j, 0)),
            pl.BlockSpec((2, 8, _HID), lambda j: (0, 0, 0)),
            pl.BlockSpec((2 * _HID, 2 * _HID), lambda j: (0, 0)),
        ],
        out_specs=pl.BlockSpec((_P_BLK, 2 * _HID), lambda j: (j, 0)),
        out_shape=jax.ShapeDtypeStruct((P, 2 * _HID), jnp.float32),
        scratch_shapes=[
            pltpu.VMEM((1, 2 * _HID), jnp.float32),
            pltpu.VMEM((1, 2), jnp.int32),
        ],
    )(x_cat, idx_cat, prm2, w2bd)


_RING = 4


def _sc_gather(gidx_p, gidx_a, table, nchunk):
    mesh = plsc.VectorSubcoreMesh(core_axis_name="c", subcore_axis_name="s")
    out_sds = jax.ShapeDtypeStruct((_NW, nchunk, _CHUNK, 2 * _HID), jnp.float32)

    @functools.partial(
        pl.kernel,
        mesh=mesh,
        out_type=[out_sds, out_sds],
        scratch_types=[
            pltpu.VMEM((nchunk, _CHUNK), jnp.int32),
            pltpu.VMEM((nchunk, _CHUNK), jnp.int32),
            pltpu.VMEM((_RING, _CHUNK, 2 * _HID), jnp.float32),
            pltpu.SemaphoreType.DMA,
            pltpu.SemaphoreType.DMA,
            pltpu.SemaphoreType.DMA,
            pltpu.SemaphoreType.DMA,
        ],
    )
    def k(gp_hbm, ga_hbm, tab_hbm, op_hbm, oa_hbm,
          idxp_v, idxa_v, rows_v, sem0, sem1, sem2, sem3):
        sems = (sem0, sem1, sem2, sem3)
        wid = lax.axis_index("s") * _NC + lax.axis_index("c")
        pltpu.sync_copy(gp_hbm.at[wid], idxp_v)
        pltpu.sync_copy(ga_hbm.at[wid], idxa_v)
        for idx_v, out in ((idxp_v, op_hbm), (idxa_v, oa_hbm)):
            def fire(c):
                return pltpu.async_copy(
                    tab_hbm.at[idx_v.at[c]], rows_v.at[c % _RING], sems[c % _RING])
            handles = {c: fire(c) for c in range(min(_RING, nchunk))}
            for c in range(nchunk):
                handles[c].wait()
                pltpu.sync_copy(rows_v.at[c % _RING], out.at[wid, c])
                if c + _RING < nchunk:
                    handles[c + _RING] = fire(c + _RING)

    return k(gidx_p, gidx_a, table)


def _node_body(gp_ref, ga_ref, cp_ref, ca_ref, xs_ref,
               w3p_ref, w4p_ref, w3a_ref, w4a_ref, wsem_ref,
               fwp_ref, fwa_ref, fws_ref, vec_ref, out_ref):
    def ln(h, g, b):
        m = jnp.mean(h, axis=-1, keepdims=True)
        v = jnp.mean((h - m) ** 2, axis=-1, keepdims=True)
        return (h - m) * lax.rsqrt(v + 1e-5) * g + b

    def branch(g_ref, c_ref, w3_ref, w4_ref, r0):
        half = slice(0, _HID) if r0 == 0 else slice(_HID, 2 * _HID)
        g = jnp.where(c_ref[...] > 0, g_ref[:, half], 0.0)
        t = jnp.dot(g, w3_ref[...], preferred_element_type=jnp.float32) \
            + vec_ref[r0:r0 + 1, :]
        t = ln(t, vec_ref[r0 + 1:r0 + 2, :], vec_ref[r0 + 2:r0 + 3, :])
        t = jnp.maximum(t, 0.0)
        return jnp.dot(t, w4_ref[...], preferred_element_type=jnp.float32) \
            + vec_ref[r0 + 3:r0 + 4, :]

    hp = branch(gp_ref, cp_ref, w3p_ref, w4p_ref, 0)
    ha = branch(ga_ref, ca_ref, w3a_ref, w4a_ref, 4)
    hs = jnp.dot(xs_ref[...], wsem_ref[...], preferred_element_type=jnp.float32) \
        + vec_ref[8:9, :]
    f = (jnp.dot(hp, fwp_ref[...], preferred_element_type=jnp.float32)
         + jnp.dot(ha, fwa_ref[...], preferred_element_type=jnp.float32)
         + jnp.dot(hs, fws_ref[...], preferred_element_type=jnp.float32)
         + vec_ref[9:10, :])
    f = ln(f, vec_ref[10:11, :], vec_ref[11:12, :])
    out_ref[...] = jnp.maximum(f, 0.0)


def _node_stage(gp, ga, cp, ca, xs, w3p, w4p, w3a, w4a, wsem, fwp, fwa, fws, vec):
    npad = gp.shape[0]
    hid = fwp.shape[1]
    sem = xs.shape[1]
    const = lambda i: (0, 0)
    return pl.pallas_call(
        _node_body,
        grid=(npad // _N_BLK,),
        in_specs=[
            pl.BlockSpec((_N_BLK, 2 * _HID), lambda i: (i, 0)),
            pl.BlockSpec((_N_BLK, 2 * _HID), lambda i: (i, 0)),
            pl.BlockSpec((_N_BLK, 1), lambda i: (i, 0)),
            pl.BlockSpec((_N_BLK, 1), lambda i: (i, 0)),
            pl.BlockSpec((_N_BLK, sem), lambda i: (i, 0)),
            pl.BlockSpec((_HID, hid), const),
            pl.BlockSpec((hid, hid), const),
            pl.BlockSpec((_HID, hid), const),
            pl.BlockSpec((hid, hid), const),
            pl.BlockSpec((sem, hid), const),
            pl.BlockSpec((hid, hid), const),
            pl.BlockSpec((hid, hid), const),
            pl.BlockSpec((hid, hid), const),
            pl.BlockSpec((16, hid), const),
        ],
        out_specs=pl.BlockSpec((_N_BLK, hid), lambda i: (i, 0)),
        out_shape=jax.ShapeDtypeStruct((npad, hid), jnp.float32),
    )(gp, ga, cp, ca, xs, w3p, w4p, w3a, w4a, wsem, fwp, fwa, fws, vec)


def _pack_point_params(p):
    return jnp.stack([
        p['w1'][:, 0], p['w1'][:, 1], p['w1'][:, 2],
        p['b1'], p['ln1_g'], p['ln1_b'], p['b2'],
        jnp.zeros_like(p['b1']),
    ], axis=0)


def kernel(x_pos, pos_batch_idx, x_aff, aff_batch_idx, x_sem, num_nodes, params):
    P = x_pos.shape[0]
    N = x_sem.shape[0]
    hid = params['sem_w'].shape[0]

    pos_idx = jnp.minimum(pos_batch_idx, num_nodes - 1).astype(jnp.int32)
    aff_idx = jnp.minimum(aff_batch_idx, num_nodes - 1).astype(jnp.int32)

    pp, pa = params['pos'], params['aff']
    x_cat = jnp.concatenate([x_pos, x_aff], axis=1)
    idx_cat = jnp.stack([pos_idx, aff_idx], axis=1)
    prm2 = jnp.stack([_pack_point_params(pp), _pack_point_params(pa)], axis=0)
    w2bd = jnp.zeros((2 * _HID, 2 * _HID), jnp.float32)
    w2bd = w2bd.at[:_HID, :_HID].set(pp['w2'].T).at[_HID:, _HID:].set(pa['w2'].T)
    table = _point_stage(x_cat, idx_cat, prm2, w2bd, P)

    nodes = jnp.arange(N, dtype=jnp.int32)
    zero = jnp.zeros((1,), jnp.int32)
    ss_p = jnp.searchsorted(pos_idx, nodes, side='right').astype(jnp.int32)
    ss_a = jnp.searchsorted(aff_idx, nodes, side='right').astype(jnp.int32)
    cnt_p = ss_p - jnp.concatenate([zero, ss_p[:-1]])
    cnt_a = ss_a - jnp.concatenate([zero, ss_a[:-1]])
    gidx_p = jnp.clip(ss_p - 1, 0, P - 1)
    gidx_a = jnp.clip(ss_a - 1, 0, P - 1)

    grp = _NW * _CHUNK
    npad = ((N + grp - 1) // grp) * grp
    nchunk = npad // grp
    pad = npad - N
    gp3 = jnp.pad(gidx_p, (0, pad)).reshape(_NW, nchunk, _CHUNK)
    ga3 = jnp.pad(gidx_a, (0, pad)).reshape(_NW, nchunk, _CHUNK)

    g_p4, g_a4 = _sc_gather(gp3, ga3, table, nchunk)
    g_p = g_p4.reshape(npad, 2 * _HID)
    g_a = g_a4.reshape(npad, 2 * _HID)

    cp = jnp.pad(cnt_p, (0, pad))[:, None]
    ca = jnp.pad(cnt_a, (0, pad))[:, None]
    xs = jnp.pad(x_sem, ((0, pad), (0, 0)))

    fw = params['fus_w']
    vec = jnp.stack([
        pp['b3'], pp['ln2_g'], pp['ln2_b'], pp['b4'],
        pa['b3'], pa['ln2_g'], pa['ln2_b'], pa['b4'],
        params['sem_b'], params['fus_b'],
        params['fus_ln_g'], params['fus_ln_b'],
        jnp.zeros((hid,), jnp.float32), jnp.zeros((hid,), jnp.float32),
        jnp.zeros((hid,), jnp.float32), jnp.zeros((hid,), jnp.float32),
    ], axis=0)

    out = _node_stage(
        g_p, g_a, cp, ca, xs,
        pp['w3'].T, pp['w4'].T, pa['w3'].T, pa['w4'].T,
        params['sem_w'].T,
        fw[:, :hid].T, fw[:, hid:2 * hid].T, fw[:, 2 * hid:].T,
        vec,
    )
    return out[:N]


# speedup vs baseline: 1.4999x; 1.4999x over previous
import functools

import jax
import jax.numpy as jnp
from jax import lax
from jax.experimental import pallas as pl
from jax.experimental.pallas import tpu as pltpu
from jax.experimental.pallas import tpu_sc as plsc

_HID = 64
_P_BLK = 4000
_N_BLK = 256

_NC = 2
_NS = 16
_NW = _NC * _NS
_CHUNK = 112


def _point_scan_body(x_ref, idx_ref, prm_ref, w2bd_ref, avg_ref, out_ref,
                     cval_ref, cidx_ref):
    B = x_ref.shape[0]

    @pl.when(pl.program_id(0) == 0)
    def _():
        cval_ref[...] = jnp.full((1, 2 * _HID), -jnp.inf, jnp.float32)
        cidx_ref[...] = jnp.full((1, 2), -1, jnp.int32)

    w16 = prm_ref[...]
    h = jnp.dot(x_ref[...], w16[0:6, :],
                preferred_element_type=jnp.float32) + w16[6:7, :]
    mm = jnp.dot(h, avg_ref[...], preferred_element_type=jnp.float32)
    d = h - mm
    vv = jnp.dot(d * d, avg_ref[...], preferred_element_type=jnp.float32)
    h = d * lax.rsqrt(vv + 1e-5) * w16[7:8, :] + w16[8:9, :]
    h = jnp.maximum(h, 0.0)
    h2 = jnp.dot(h, w2bd_ref[...], preferred_element_type=jnp.float32) \
        + w16[9:10, :]
    h0, h1 = h2[:, :_HID], h2[:, _HID:]

    idx = idx_ref[...]
    i0, i1 = idx[:, 0:1], idx[:, 1:2]
    row0 = lax.broadcasted_iota(jnp.int32, (B, 1), 0) == 0
    h0 = jnp.where(row0 & (i0 == cidx_ref[:, 0:1]),
                   jnp.maximum(h0, cval_ref[:, :_HID]), h0)
    h1 = jnp.where(row0 & (i1 == cidx_ref[:, 1:2]),
                   jnp.maximum(h1, cval_ref[:, _HID:]), h1)

    s = 1
    while s < B:
        eq = idx == jnp.concatenate(
            [jnp.full((s, 2), -1, jnp.int32), idx[:B - s, :]], axis=0)
        neg = jnp.full((s, _HID), -jnp.inf, jnp.float32)
        h0_sh = jnp.concatenate([neg, h0[:B - s, :]], axis=0)
        h1_sh = jnp.concatenate([neg, h1[:B - s, :]], axis=0)
        h0 = jnp.where(eq[:, 0:1], jnp.maximum(h0, h0_sh), h0)
        h1 = jnp.where(eq[:, 1:2], jnp.maximum(h1, h1_sh), h1)
        s *= 2

    res = jnp.concatenate([h0, h1], axis=1)
    out_ref[...] = res
    cval_ref[...] = res[B - 1:B, :]
    cidx_ref[...] = idx[B - 1:B, :]


def _point_stage(x_cat, idx_cat, prm2, w2bd, avg, P):
    return pl.pallas_call(
        _point_scan_body,
        grid=(P // _P_BLK,),
        in_specs=[
            pl.BlockSpec((_P_BLK, 6), lambda j: (j, 0)),
            pl.BlockSpec((_P_BLK, 2), lambda j: (j, 0)),
            pl.BlockSpec((16, 2 * _HID), lambda j: (0, 0)),
            pl.BlockSpec((2 * _HID, 2 * _HID), lambda j: (0, 0)),
            pl.BlockSpec((2 * _HID, 2 * _HID), lambda j: (0, 0)),
        ],
        out_specs=pl.BlockSpec((_P_BLK, 2 * _HID), lambda j: (j, 0)),
        out_shape=jax.ShapeDtypeStruct((P, 2 * _HID), jnp.float32),
        scratch_shapes=[
            pltpu.VMEM((1, 2 * _HID), jnp.float32),
            pltpu.VMEM((1, 2), jnp.int32),
        ],
    )(x_cat, idx_cat, prm2, w2bd, avg)


_RING = 4


def _sc_gather(gidx_p, gidx_a, table, nchunk):
    mesh = plsc.VectorSubcoreMesh(core_axis_name="c", subcore_axis_name="s")
    out_sds = jax.ShapeDtypeStruct((_NW, nchunk, _CHUNK, 2 * _HID), jnp.float32)

    @functools.partial(
        pl.kernel,
        mesh=mesh,
        out_type=[out_sds, out_sds],
        scratch_types=[
            pltpu.VMEM((nchunk, _CHUNK), jnp.int32),
            pltpu.VMEM((nchunk, _CHUNK), jnp.int32),
            pltpu.VMEM((_RING, _CHUNK, 2 * _HID), jnp.float32),
            pltpu.SemaphoreType.DMA,
            pltpu.SemaphoreType.DMA,
            pltpu.SemaphoreType.DMA,
            pltpu.SemaphoreType.DMA,
        ],
    )
    def k(gp_hbm, ga_hbm, tab_hbm, op_hbm, oa_hbm,
          idxp_v, idxa_v, rows_v, sem0, sem1, sem2, sem3):
        sems = (sem0, sem1, sem2, sem3)
        wid = lax.axis_index("s") * _NC + lax.axis_index("c")
        pltpu.sync_copy(gp_hbm.at[wid], idxp_v)
        pltpu.sync_copy(ga_hbm.at[wid], idxa_v)
        for idx_v, out in ((idxp_v, op_hbm), (idxa_v, oa_hbm)):
            def fire(c):
                return pltpu.async_copy(
                    tab_hbm.at[idx_v.at[c]], rows_v.at[c % _RING], sems[c % _RING])
            handles = {c: fire(c) for c in range(min(_RING, nchunk))}
            for c in range(nchunk):
                handles[c].wait()
                pltpu.sync_copy(rows_v.at[c % _RING], out.at[wid, c])
                if c + _RING < nchunk:
                    handles[c + _RING] = fire(c + _RING)

    return k(gidx_p, gidx_a, table)


def _node_body(gp_ref, ga_ref, cp_ref, ca_ref, xs_ref,
               w3p_ref, w4p_ref, w3a_ref, w4a_ref, wsem_ref,
               fwp_ref, fwa_ref, fws_ref, vec_ref, out_ref):
    def ln(h, g, b):
        m = jnp.mean(h, axis=-1, keepdims=True)
        v = jnp.mean((h - m) ** 2, axis=-1, keepdims=True)
        return (h - m) * lax.rsqrt(v + 1e-5) * g + b

    def branch(g_ref, c_ref, w3_ref, w4_ref, r0):
        half = slice(0, _HID) if r0 == 0 else slice(_HID, 2 * _HID)
        g = jnp.where(c_ref[...] > 0, g_ref[:, half], 0.0)
        t = jnp.dot(g, w3_ref[...], preferred_element_type=jnp.float32) \
            + vec_ref[r0:r0 + 1, :]
        t = ln(t, vec_ref[r0 + 1:r0 + 2, :], vec_ref[r0 + 2:r0 + 3, :])
        t = jnp.maximum(t, 0.0)
        return jnp.dot(t, w4_ref[...], preferred_element_type=jnp.float32) \
            + vec_ref[r0 + 3:r0 + 4, :]

    hp = branch(gp_ref, cp_ref, w3p_ref, w4p_ref, 0)
    ha = branch(ga_ref, ca_ref, w3a_ref, w4a_ref, 4)
    hs = jnp.dot(xs_ref[...], wsem_ref[...], preferred_element_type=jnp.float32) \
        + vec_ref[8:9, :]
    f = (jnp.dot(hp, fwp_ref[...], preferred_element_type=jnp.float32)
         + jnp.dot(ha, fwa_ref[...], preferred_element_type=jnp.float32)
         + jnp.dot(hs, fws_ref[...], preferred_element_type=jnp.float32)
         + vec_ref[9:10, :])
    f = ln(f, vec_ref[10:11, :], vec_ref[11:12, :])
    out_ref[...] = jnp.maximum(f, 0.0)


def _node_stage(gp, ga, cp, ca, xs, w3p, w4p, w3a, w4a, wsem, fwp, fwa, fws, vec):
    npad = gp.shape[0]
    hid = fwp.shape[1]
    sem = xs.shape[1]
    const = lambda i: (0, 0)
    return pl.pallas_call(
        _node_body,
        grid=(npad // _N_BLK,),
        in_specs=[
            pl.BlockSpec((_N_BLK, 2 * _HID), lambda i: (i, 0)),
            pl.BlockSpec((_N_BLK, 2 * _HID), lambda i: (i, 0)),
            pl.BlockSpec((_N_BLK, 1), lambda i: (i, 0)),
            pl.BlockSpec((_N_BLK, 1), lambda i: (i, 0)),
            pl.BlockSpec((_N_BLK, sem), lambda i: (i, 0)),
            pl.BlockSpec((_HID, hid), const),
            pl.BlockSpec((hid, hid), const),
            pl.BlockSpec((_HID, hid), const),
            pl.BlockSpec((hid, hid), const),
            pl.BlockSpec((sem, hid), const),
            pl.BlockSpec((hid, hid), const),
            pl.BlockSpec((hid, hid), const),
            pl.BlockSpec((hid, hid), const),
            pl.BlockSpec((16, hid), const),
        ],
        out_specs=pl.BlockSpec((_N_BLK, hid), lambda i: (i, 0)),
        out_shape=jax.ShapeDtypeStruct((npad, hid), jnp.float32),
    )(gp, ga, cp, ca, xs, w3p, w4p, w3a, w4a, wsem, fwp, fwa, fws, vec)


def _pack_point_params(pp, pa):
    z = jnp.zeros((_HID,), jnp.float32)
    rows = [
        jnp.concatenate([pp['w1'][:, 0], z]),
        jnp.concatenate([pp['w1'][:, 1], z]),
        jnp.concatenate([pp['w1'][:, 2], z]),
        jnp.concatenate([z, pa['w1'][:, 0]]),
        jnp.concatenate([z, pa['w1'][:, 1]]),
        jnp.concatenate([z, pa['w1'][:, 2]]),
        jnp.concatenate([pp['b1'], pa['b1']]),
        jnp.concatenate([pp['ln1_g'], pa['ln1_g']]),
        jnp.concatenate([pp['ln1_b'], pa['ln1_b']]),
        jnp.concatenate([pp['b2'], pa['b2']]),
    ]
    rows += [jnp.zeros((2 * _HID,), jnp.float32)] * 6
    return jnp.stack(rows, axis=0)


def kernel(x_pos, pos_batch_idx, x_aff, aff_batch_idx, x_sem, num_nodes, params):
    P = x_pos.shape[0]
    N = x_sem.shape[0]
    hid = params['sem_w'].shape[0]

    pos_idx = jnp.minimum(pos_batch_idx, num_nodes - 1).astype(jnp.int32)
    aff_idx = jnp.minimum(aff_batch_idx, num_nodes - 1).astype(jnp.int32)

    pp, pa = params['pos'], params['aff']
    x_cat = jnp.concatenate([x_pos, x_aff], axis=1)
    idx_cat = jnp.stack([pos_idx, aff_idx], axis=1)
    prm2 = _pack_point_params(pp, pa)
    w2bd = jnp.zeros((2 * _HID, 2 * _HID), jnp.float32)
    w2bd = w2bd.at[:_HID, :_HID].set(pp['w2'].T).at[_HID:, _HID:].set(pa['w2'].T)
    blk = jnp.ones((_HID, _HID), jnp.float32) / _HID
    zb = jnp.zeros((_HID, _HID), jnp.float32)
    avg = jnp.block([[blk, zb], [zb, blk]])
    table = _point_stage(x_cat, idx_cat, prm2, w2bd, avg, P)

    nodes = jnp.arange(N, dtype=jnp.int32)
    zero = jnp.zeros((1,), jnp.int32)
    ss_p = jnp.searchsorted(pos_idx, nodes, side='right').astype(jnp.int32)
    ss_a = jnp.searchsorted(aff_idx, nodes, side='right').astype(jnp.int32)
    cnt_p = ss_p - jnp.concatenate([zero, ss_p[:-1]])
    cnt_a = ss_a - jnp.concatenate([zero, ss_a[:-1]])
    gidx_p = jnp.clip(ss_p - 1, 0, P - 1)
    gidx_a = jnp.clip(ss_a - 1, 0, P - 1)

    grp = _NW * _CHUNK
    npad = ((N + grp - 1) // grp) * grp
    nchunk = npad // grp
    pad = npad - N
    gp3 = jnp.pad(gidx_p, (0, pad)).reshape(_NW, nchunk, _CHUNK)
    ga3 = jnp.pad(gidx_a, (0, pad)).reshape(_NW, nchunk, _CHUNK)

    g_p4, g_a4 = _sc_gather(gp3, ga3, table, nchunk)
    g_p = g_p4.reshape(npad, 2 * _HID)
    g_a = g_a4.reshape(npad, 2 * _HID)

    cp = jnp.pad(cnt_p, (0, pad))[:, None]
    ca = jnp.pad(cnt_a, (0, pad))[:, None]
    xs = jnp.pad(x_sem, ((0, pad), (0, 0)))

    fw = params['fus_w']
    vec = jnp.stack([
        pp['b3'], pp['ln2_g'], pp['ln2_b'], pp['b4'],
        pa['b3'], pa['ln2_g'], pa['ln2_b'], pa['b4'],
        params['sem_b'], params['fus_b'],
        params['fus_ln_g'], params['fus_ln_b'],
        jnp.zeros((hid,), jnp.float32), jnp.zeros((hid,), jnp.float32),
        jnp.zeros((hid,), jnp.float32), jnp.zeros((hid,), jnp.float32),
    ], axis=0)

    out = _node_stage(
        g_p, g_a, cp, ca, xs,
        pp['w3'].T, pp['w4'].T, pa['w3'].T, pa['w4'].T,
        params['sem_w'].T,
        fw[:, :hid].T, fw[:, hid:2 * hid].T, fw[:, 2 * hid:].T,
        vec,
    )
    return out[:N]
